# baseline (speedup 1.0000x reference)
import jax
import jax.numpy as jnp
from jax.experimental import pallas as pl
from jax.experimental.pallas import tpu as pltpu

N = 8192
D = 6370
NBLK = 32
BM = N // NBLK
INT32_MIN = -2147483648


def _w2_copy(w2_hbm, w2buf, sem, jj):
    return pltpu.make_async_copy(
        w2_hbm.at[:, pl.ds(jj * BM, BM)],
        w2buf.at[jj % 2],
        sem.at[jj % 2])


def _fused_kern(x_ref, w1_ref, b1_ref, w2_hbm, b2_ref, score_ref, g_ref,
                o_ref, pen_ref, s_acc, w2buf, sem):
    j = pl.program_id(0)

    @pl.when(j == 0)
    def _():
        _w2_copy(w2_hbm, w2buf, sem, 0).start()

    @pl.when(j + 1 < NBLK)
    def _():
        _w2_copy(w2_hbm, w2buf, sem, j + 1).start()

    h = jax.lax.dot_general(
        x_ref[...], w1_ref[...],
        dimension_numbers=(((1,), (1,)), ((), ())),
        preferred_element_type=jnp.float32)
    h = jnp.maximum(h + b1_ref[...], 0.0)

    _w2_copy(w2_hbm, w2buf, sem, j).wait()
    part = jax.lax.dot_general(
        h, w2buf[j % 2],
        dimension_numbers=(((1,), (1,)), ((), ())),
        preferred_element_type=jnp.float32)

    @pl.when(j == 0)
    def _():
        s_acc[...] = b2_ref[...]

    s_acc[...] += part

    @pl.when(j == NBLK - 1)
    def _():
        z = s_acc[...] + g_ref[...]
        m = jnp.max(z)
        ssum = jnp.sum(jnp.exp(z - m))
        y_max = 1.0 / ssum
        c = (1.0 - y_max) + y_max
        iota = jax.lax.broadcasted_iota(jnp.int32, z.shape, 1)
        idx = jnp.min(jnp.where(z == m, iota, N))
        kk = idx + 1

        u = jax.lax.bitcast_convert_type(score_ref[...], jnp.int32)
        key = jnp.where(u >= 0, u, jnp.int32(INT32_MIN) - u)

        def cnt_ge(t):
            return jnp.sum((key >= t).astype(jnp.int32))

        t0 = jnp.where(cnt_ge(jnp.int32(0)) >= kk,
                       jnp.int32(0), jnp.int32(INT32_MIN))

        def bit_body(i, t):
            tp = t + (jnp.int32(1) << (jnp.int32(30) - i))
            return jnp.where(cnt_ge(tp) >= kk, tp, t)

        t = jax.lax.fori_loop(0, 31, bit_body, t0)

        gt = key > t
        eq = key == t
        count_gt = jnp.sum(gt.astype(jnp.int32))
        need = kk - count_gt
        eqi = eq.astype(jnp.int32)

        def idx_body(_, lohi):
            lo, hi = lohi
            mid = (lo + hi) // 2
            ok = jnp.sum(jnp.where(iota <= mid, eqi, 0)) >= need
            return (jnp.where(ok, lo, mid + 1), jnp.where(ok, mid, hi))

        lo, _ = jax.lax.fori_loop(0, 13, idx_body,
                                  (jnp.int32(0), jnp.int32(N - 1)))

        keep = gt | (eq & (iota <= lo))
        o_ref[...] = score_ref[...] * jnp.where(keep, c, 0.0)
        pen_ref[...] = jnp.full((1, 1), c * kk.astype(jnp.float32),
                                jnp.float32)


def kernel(score_vector, condition, W1, b1, W2, b2):
    u = jax.random.uniform(jax.random.key(1234), (1, N),
                           minval=1e-10, maxval=1.0)
    g = -jnp.log(-jnp.log(u))

    out, pen = pl.pallas_call(
        _fused_kern,
        grid=(NBLK,),
        in_specs=[
            pl.BlockSpec((1, D), lambda j: (0, 0)),
            pl.BlockSpec((BM, D), lambda j: (j, 0)),
            pl.BlockSpec((1, BM), lambda j: (0, j)),
            pl.BlockSpec(memory_space=pltpu.MemorySpace.HBM),
            pl.BlockSpec((1, N), lambda j: (0, 0)),
            pl.BlockSpec((1, N), lambda j: (0, 0)),
            pl.BlockSpec((1, N), lambda j: (0, 0)),
        ],
        out_specs=(pl.BlockSpec((1, N), lambda j: (0, 0)),
                   pl.BlockSpec((1, 1), lambda j: (0, 0))),
        out_shape=(jax.ShapeDtypeStruct((1, N), jnp.float32),
                   jax.ShapeDtypeStruct((1, 1), jnp.float32)),
        scratch_shapes=[pltpu.VMEM((1, N), jnp.float32),
                        pltpu.VMEM((2, N, BM), jnp.float32),
                        pltpu.SemaphoreType.DMA((2,))],
    )(condition, W1, b1.reshape(1, N), W2, b2.reshape(1, N),
      score_vector, g)
    return out, pen.reshape(1)
